# baseline (speedup 1.0000x reference)
import functools

import jax
import jax.numpy as jnp
from jax import lax
from jax.experimental import pallas as pl
from jax.experimental.pallas import tpu as pltpu
from jax.experimental.pallas import tpu_sc as plsc

V = 1000000
D = 64
H = D // 2
B = 16384
LANES = 128

NC = 2
NS = 16
NW = NC * NS
B_SC = 10240
B_TC = B - B_SC
B_PER_W = B_SC // NW
GRP = 4
MG = 8


def _sc_gather_t(table_t, ids):
    ids2 = ids.reshape(NW, B_PER_W)
    mesh = plsc.VectorSubcoreMesh(core_axis_name="c", subcore_axis_name="s")

    @functools.partial(
        pl.kernel,
        mesh=mesh,
        out_type=jax.ShapeDtypeStruct((B_SC, D), jnp.float32),
        scratch_types=[
            pltpu.VMEM((B_PER_W,), jnp.int32),
            pltpu.VMEM((GRP, D, LANES), jnp.float32),
            pltpu.VMEM((GRP, D, LANES), jnp.float32),
            pltpu.VMEM((GRP, D, LANES), jnp.float32),
            pltpu.VMEM((16, D), jnp.float32),
            pltpu.SemaphoreType.DMA,
            pltpu.SemaphoreType.DMA,
            pltpu.SemaphoreType.DMA,
        ],
        compiler_params=pltpu.CompilerParams(needs_layout_passes=False),
    )
    def k(table_hbm, idx_hbm, out_hbm, idx_s, buf0, buf1, buf2, rows_v,
          sem0, sem1, sem2):
        wid = lax.axis_index("s") * NC + lax.axis_index("c")
        pltpu.sync_copy(idx_hbm.at[wid], idx_s)

        jiota = lax.iota(jnp.int32, 16)

        def fire(grp_ids, buf, sem):
            for l, idv in enumerate(grp_ids):
                blk = pl.multiple_of((idv >> 7) << 7, LANES)
                pltpu.async_copy(
                    table_hbm.at[:, pl.ds(blk, LANES)],
                    buf.at[l],
                    sem,
                )

        def drain(buf, sem):
            for l in range(GRP):
                pltpu.make_async_copy(
                    table_hbm.at[:, pl.ds(0, LANES)],
                    buf.at[l],
                    sem,
                ).wait()

        def extract(base_k, grp_ids, buf):
            for l, idv in enumerate(grp_ids):
                col = jnp.broadcast_to(idv & 127, (16,))
                kv = jnp.broadcast_to(jnp.int32(base_k + l), (16,))
                lv = jnp.broadcast_to(jnp.int32(l), (16,))
                for q in range(D // 16):
                    jv = jiota + (16 * q)
                    x = plsc.load_gather(buf, [lv, jv, col])
                    plsc.store_scatter(rows_v, [kv, jv], x)

        def body(s, _):
            base = s * 16
            vec = idx_s[pl.ds(base, 16)]
            grp = [[vec[4 * g + l] for l in range(GRP)] for g in range(4)]
            fire(grp[0], buf0, sem0)
            fire(grp[1], buf1, sem1)
            fire(grp[2], buf2, sem2)
            drain(buf0, sem0)
            extract(0, grp[0], buf0)
            fire(grp[3], buf0, sem0)
            drain(buf1, sem1)
            extract(4, grp[1], buf1)
            drain(buf2, sem2)
            extract(8, grp[2], buf2)
            drain(buf0, sem0)
            extract(12, grp[3], buf0)
            pltpu.sync_copy(rows_v, out_hbm.at[pl.ds(wid * B_PER_W + base, 16)])
            return ()

        lax.fori_loop(0, B_PER_W // 16, body, (), unroll=False)

    return k(table_t, ids2)


def _tc_gather_body(ids_ref, *refs):
    xs, o_ref = refs[:MG], refs[MG]
    i = pl.program_id(0)
    cols = []
    lane = lax.broadcasted_iota(jnp.int32, (1, LANES), 1)
    for q in range(MG):
        c = ids_ref[MG * i + q] & 127
        onehot = (lane == c).astype(jnp.float32)
        x = xs[q][...]
        cols.append(
            jnp.dot(x, onehot.T, preferred_element_type=jnp.float32)
        )
    o_ref[...] = jnp.concatenate(cols, axis=1).T


def _tc_gather(table_t, ids_tc):
    grid = B_TC // MG
    spec_tab = [
        pl.BlockSpec(
            (D, LANES),
            functools.partial(
                lambda q, i, ids_ref: (0, ids_ref[MG * i + q] >> 7), q
            ),
        )
        for q in range(MG)
    ]
    return pl.pallas_call(
        _tc_gather_body,
        grid_spec=pltpu.PrefetchScalarGridSpec(
            num_scalar_prefetch=1,
            grid=(grid,),
            in_specs=spec_tab,
            out_specs=pl.BlockSpec((MG, D), lambda i, ids_ref: (i, 0)),
        ),
        out_shape=jax.ShapeDtypeStruct((B_TC, D), jnp.float32),
    )(ids_tc, *([table_t] * MG))


def _mlp_body(x_ref, w1_ref, b1_ref, w2_ref, b2_ref, o1_ref, o2_ref, o3_ref):
    x = x_ref[...]
    h = jnp.dot(x, w1_ref[...], preferred_element_type=jnp.float32) + b1_ref[...]
    h = jnp.maximum(h, 0.0)
    out3 = jnp.dot(h, w2_ref[...], preferred_element_type=jnp.float32) + b2_ref[...]
    o1_ref[...] = out3[:, 0]
    o2_ref[...] = out3[:, 1]
    o3_ref[...] = out3[:, 2]


def _tc_mlp(emb, w1c, b1c, w2blk, b2c):
    blk = 2048
    grid = B // blk
    head = jax.ShapeDtypeStruct((B,), jnp.float32)
    return pl.pallas_call(
        _mlp_body,
        grid=(grid,),
        in_specs=[
            pl.BlockSpec((blk, D), lambda i: (i, 0)),
            pl.BlockSpec((D, 3 * H), lambda i: (0, 0)),
            pl.BlockSpec((1, 3 * H), lambda i: (0, 0)),
            pl.BlockSpec((3 * H, 3), lambda i: (0, 0)),
            pl.BlockSpec((1, 3), lambda i: (0, 0)),
        ],
        out_specs=[pl.BlockSpec((blk,), lambda i: (i,))] * 3,
        out_shape=[head, head, head],
    )(emb, w1c, b1c, w2blk, b2c)


def kernel(instrument_ids, table, vW1, vb1, vW2, vb2,
           lW1, lb1, lW2, lb2, tW1, tb1, tW2, tb2):
    ids = instrument_ids.astype(jnp.int32)
    table_t = table.T
    emb_sc = _sc_gather_t(table_t, ids[:B_SC])
    emb_tc = _tc_gather(table_t, ids[B_SC:])
    embeddings = jnp.concatenate([emb_sc, emb_tc], axis=0)

    w1c = jnp.concatenate([vW1, lW1, tW1], axis=1)
    b1c = jnp.concatenate([vb1, lb1, tb1], axis=0)[None, :]
    zero = jnp.zeros((H, 1), jnp.float32)
    w2blk = jnp.concatenate(
        [
            jnp.concatenate([vW2, zero, zero], axis=1),
            jnp.concatenate([zero, lW2, zero], axis=1),
            jnp.concatenate([zero, zero, tW2], axis=1),
        ],
        axis=0,
    )
    b2c = jnp.concatenate([vb2, lb2, tb2], axis=0)[None, :]

    vol, liq, trd = _tc_mlp(embeddings, w1c, b1c, w2blk, b2c)
    return (embeddings, vol[:, None], liq[:, None], trd[:, None])


# speedup vs baseline: 1.8494x; 1.8494x over previous
import functools

import jax
import jax.numpy as jnp
from jax import lax
from jax.experimental import pallas as pl
from jax.experimental.pallas import tpu as pltpu
from jax.experimental.pallas import tpu_sc as plsc

V = 1000000
D = 64
H = D // 2
B = 16384
LANES = 128

NC = 2
NS = 16
NW = NC * NS
B_SC = B
B_PER_W = B_SC // NW
GRP = 4


def _sc_gather_t(table_t, ids):
    ids2 = ids.reshape(NW, B_PER_W)
    mesh = plsc.VectorSubcoreMesh(core_axis_name="c", subcore_axis_name="s")

    @functools.partial(
        pl.kernel,
        mesh=mesh,
        out_type=jax.ShapeDtypeStruct((B_SC, D), jnp.float32),
        scratch_types=[
            pltpu.VMEM((B_PER_W,), jnp.int32),
            pltpu.VMEM((GRP, D, LANES), jnp.float32),
            pltpu.VMEM((GRP, D, LANES), jnp.float32),
            pltpu.VMEM((GRP, D, LANES), jnp.float32),
            pltpu.VMEM((16, D), jnp.float32),
            pltpu.SemaphoreType.DMA,
            pltpu.SemaphoreType.DMA,
            pltpu.SemaphoreType.DMA,
        ],
        compiler_params=pltpu.CompilerParams(needs_layout_passes=False),
    )
    def k(table_hbm, idx_hbm, out_hbm, idx_s, buf0, buf1, buf2, rows_v,
          sem0, sem1, sem2):
        wid = lax.axis_index("s") * NC + lax.axis_index("c")
        pltpu.sync_copy(idx_hbm.at[wid], idx_s)

        jiota = lax.iota(jnp.int32, 16)

        def fire(grp_ids, buf, sem):
            for l, idv in enumerate(grp_ids):
                blk = pl.multiple_of((idv >> 7) << 7, LANES)
                for t in range(8):
                    pltpu.async_copy(
                        table_hbm.at[pl.ds(8 * t, 8), pl.ds(blk, LANES)],
                        buf.at[l].at[pl.ds(8 * t, 8)],
                        sem,
                    )

        def drain(buf, sem):
            for l in range(GRP):
                pltpu.make_async_copy(
                    table_hbm.at[:, pl.ds(0, LANES)],
                    buf.at[l],
                    sem,
                ).wait()

        def extract(base_k, grp_ids, buf):
            for l, idv in enumerate(grp_ids):
                col = jnp.broadcast_to(idv & 127, (16,))
                kv = jnp.broadcast_to(jnp.int32(base_k + l), (16,))
                lv = jnp.broadcast_to(jnp.int32(l), (16,))
                for q in range(D // 16):
                    jv = jiota + (16 * q)
                    x = plsc.load_gather(buf, [lv, jv, col])
                    plsc.store_scatter(rows_v, [kv, jv], x)

        def body(s, _):
            base = s * 16
            vec = idx_s[pl.ds(base, 16)]
            grp = [[vec[4 * g + l] for l in range(GRP)] for g in range(4)]
            fire(grp[0], buf0, sem0)
            fire(grp[1], buf1, sem1)
            fire(grp[2], buf2, sem2)
            drain(buf0, sem0)
            extract(0, grp[0], buf0)
            fire(grp[3], buf0, sem0)
            drain(buf1, sem1)
            extract(4, grp[1], buf1)
            drain(buf2, sem2)
            extract(8, grp[2], buf2)
            drain(buf0, sem0)
            extract(12, grp[3], buf0)
            pltpu.sync_copy(rows_v, out_hbm.at[pl.ds(wid * B_PER_W + base, 16)])
            return ()

        lax.fori_loop(0, B_PER_W // 16, body, (), unroll=False)

    return k(table_t, ids2)


def _mlp_body(x_ref, w1_ref, b1_ref, w2_ref, b2_ref, o1_ref, o2_ref, o3_ref):
    x = x_ref[...]
    h = jnp.dot(x, w1_ref[...], preferred_element_type=jnp.float32) + b1_ref[...]
    h = jnp.maximum(h, 0.0)
    out3 = jnp.dot(h, w2_ref[...], preferred_element_type=jnp.float32) + b2_ref[...]
    o1_ref[...] = out3[:, 0]
    o2_ref[...] = out3[:, 1]
    o3_ref[...] = out3[:, 2]


def _tc_mlp(emb, w1c, b1c, w2blk, b2c):
    blk = 2048
    grid = B // blk
    head = jax.ShapeDtypeStruct((B,), jnp.float32)
    return pl.pallas_call(
        _mlp_body,
        grid=(grid,),
        in_specs=[
            pl.BlockSpec((blk, D), lambda i: (i, 0)),
            pl.BlockSpec((D, 3 * H), lambda i: (0, 0)),
            pl.BlockSpec((1, 3 * H), lambda i: (0, 0)),
            pl.BlockSpec((3 * H, 3), lambda i: (0, 0)),
            pl.BlockSpec((1, 3), lambda i: (0, 0)),
        ],
        out_specs=[pl.BlockSpec((blk,), lambda i: (i,))] * 3,
        out_shape=[head, head, head],
    )(emb, w1c, b1c, w2blk, b2c)


def kernel(instrument_ids, table, vW1, vb1, vW2, vb2,
           lW1, lb1, lW2, lb2, tW1, tb1, tW2, tb2):
    ids = instrument_ids.astype(jnp.int32)
    embeddings = _sc_gather_t(table.T, ids)

    w1c = jnp.concatenate([vW1, lW1, tW1], axis=1)
    b1c = jnp.concatenate([vb1, lb1, tb1], axis=0)[None, :]
    zero = jnp.zeros((H, 1), jnp.float32)
    w2blk = jnp.concatenate(
        [
            jnp.concatenate([vW2, zero, zero], axis=1),
            jnp.concatenate([zero, lW2, zero], axis=1),
            jnp.concatenate([zero, zero, tW2], axis=1),
        ],
        axis=0,
    )
    b2c = jnp.concatenate([vb2, lb2, tb2], axis=0)[None, :]

    vol, liq, trd = _tc_mlp(embeddings, w1c, b1c, w2blk, b2c)
    return (embeddings, vol[:, None], liq[:, None], trd[:, None])


# speedup vs baseline: 1.8541x; 1.0026x over previous
import functools

import jax
import jax.numpy as jnp
from jax import lax
from jax.experimental import pallas as pl
from jax.experimental.pallas import tpu as pltpu
from jax.experimental.pallas import tpu_sc as plsc

V = 1000000
D = 64
H = D // 2
B = 16384
LANES = 128

NC = 2
NS = 16
NW = NC * NS
B_SC = B
B_PER_W = B_SC // NW
GRP = 4


def _sc_gather_t(table_t, ids):
    ids2 = ids.reshape(NW, B_PER_W)
    mesh = plsc.VectorSubcoreMesh(core_axis_name="c", subcore_axis_name="s")

    @functools.partial(
        pl.kernel,
        mesh=mesh,
        out_type=jax.ShapeDtypeStruct((B_SC, D), jnp.float32),
        scratch_types=[
            pltpu.VMEM((B_PER_W,), jnp.int32),
            pltpu.VMEM((GRP, D, LANES), jnp.float32),
            pltpu.VMEM((GRP, D, LANES), jnp.float32),
            pltpu.VMEM((GRP, D, LANES), jnp.float32),
            pltpu.VMEM((16, D), jnp.float32),
            pltpu.SemaphoreType.DMA,
            pltpu.SemaphoreType.DMA,
            pltpu.SemaphoreType.DMA,
        ],
        compiler_params=pltpu.CompilerParams(needs_layout_passes=False),
    )
    def k(table_hbm, idx_hbm, out_hbm, idx_s, buf0, buf1, buf2, rows_v,
          sem0, sem1, sem2):
        wid = lax.axis_index("s") * NC + lax.axis_index("c")
        pltpu.sync_copy(idx_hbm.at[wid], idx_s)

        jiota = lax.iota(jnp.int32, 16)

        def fire(grp_ids, buf, sem):
            for l, idv in enumerate(grp_ids):
                blk = pl.multiple_of((idv >> 7) << 7, LANES)
                for t in range(8):
                    pltpu.async_copy(
                        table_hbm.at[pl.ds(8 * t, 8), pl.ds(blk, LANES)],
                        buf.at[l].at[pl.ds(8 * t, 8)],
                        sem,
                    )

        def drain(buf, sem):
            for l in range(GRP):
                pltpu.make_async_copy(
                    table_hbm.at[:, pl.ds(0, LANES)],
                    buf.at[l],
                    sem,
                ).wait()

        def extract(base_k, grp_ids, buf):
            for l, idv in enumerate(grp_ids):
                col = jnp.broadcast_to(idv & 127, (16,))
                kv = jnp.broadcast_to(jnp.int32(base_k + l), (16,))
                lv = jnp.broadcast_to(jnp.int32(l), (16,))
                for q in range(D // 16):
                    jv = jiota + (16 * q)
                    x = plsc.load_gather(buf, [lv, jv, col])
                    plsc.store_scatter(rows_v, [kv, jv], x)

        def body(s, _):
            base = s * 16
            vec = idx_s[pl.ds(base, 16)]
            grp = [[vec[4 * g + l] for l in range(GRP)] for g in range(4)]
            fire(grp[0], buf0, sem0)
            fire(grp[1], buf1, sem1)
            fire(grp[2], buf2, sem2)
            drain(buf0, sem0)
            extract(0, grp[0], buf0)
            fire(grp[3], buf0, sem0)
            drain(buf1, sem1)
            extract(4, grp[1], buf1)
            drain(buf2, sem2)
            extract(8, grp[2], buf2)
            drain(buf0, sem0)
            extract(12, grp[3], buf0)
            pltpu.sync_copy(rows_v, out_hbm.at[pl.ds(wid * B_PER_W + base, 16)])
            return ()

        lax.fori_loop(0, B_PER_W // 16, body, (), unroll=False)

    return k(table_t, ids2)


def _mlp_body(x_ref, w1_ref, b1_ref, w2_ref, b2_ref, o1_ref, o2_ref, o3_ref):
    x = x_ref[...]
    h = jnp.dot(x, w1_ref[...], preferred_element_type=jnp.float32) + b1_ref[...]
    h = jnp.maximum(h, 0.0)
    out3 = jnp.dot(h, w2_ref[...], preferred_element_type=jnp.float32) + b2_ref[...]
    o1_ref[...] = out3[:, 0]
    o2_ref[...] = out3[:, 1]
    o3_ref[...] = out3[:, 2]


def _tc_mlp(emb, w1c, b1c, w2blk, b2c):
    blk = 4096
    grid = B // blk
    head = jax.ShapeDtypeStruct((B,), jnp.float32)
    return pl.pallas_call(
        _mlp_body,
        grid=(grid,),
        in_specs=[
            pl.BlockSpec((blk, D), lambda i: (i, 0)),
            pl.BlockSpec((D, 3 * H), lambda i: (0, 0)),
            pl.BlockSpec((1, 3 * H), lambda i: (0, 0)),
            pl.BlockSpec((3 * H, 3), lambda i: (0, 0)),
            pl.BlockSpec((1, 3), lambda i: (0, 0)),
        ],
        out_specs=[pl.BlockSpec((blk,), lambda i: (i,))] * 3,
        out_shape=[head, head, head],
    )(emb, w1c, b1c, w2blk, b2c)


def kernel(instrument_ids, table, vW1, vb1, vW2, vb2,
           lW1, lb1, lW2, lb2, tW1, tb1, tW2, tb2):
    ids = instrument_ids.astype(jnp.int32)
    embeddings = _sc_gather_t(table.T, ids)

    w1c = jnp.concatenate([vW1, lW1, tW1], axis=1)
    b1c = jnp.concatenate([vb1, lb1, tb1], axis=0)[None, :]
    zero = jnp.zeros((H, 1), jnp.float32)
    w2blk = jnp.concatenate(
        [
            jnp.concatenate([vW2, zero, zero], axis=1),
            jnp.concatenate([zero, lW2, zero], axis=1),
            jnp.concatenate([zero, zero, tW2], axis=1),
        ],
        axis=0,
    )
    b2c = jnp.concatenate([vb2, lb2, tb2], axis=0)[None, :]

    vol, liq, trd = _tc_mlp(embeddings, w1c, b1c, w2blk, b2c)
    return (embeddings, vol[:, None], liq[:, None], trd[:, None])
